# baseline (speedup 1.0000x reference)
import functools

import jax
import jax.numpy as jnp
from jax import lax
from jax.experimental import pallas as pl
from jax.experimental.pallas import tpu as pltpu
from jax.experimental.pallas import tpu_sc as plsc

_LANES = 16
_NC, _NS = 2, 16
_NW = _NC * _NS
_CHUNK = 32


def _sc_body(x_hbm, pos_hbm, table_hbm, out_hbm, idx_v, in_v, pe_v, sem):
    wid = lax.axis_index("s") * _NC + lax.axis_index("c")
    n_chunks = pos_hbm.shape[1]
    d = x_hbm.shape[1]
    n_vecs = d // _LANES
    pltpu.sync_copy(pos_hbm.at[wid], idx_v)
    for c in range(n_chunks):
        row0 = (wid * n_chunks + c) * _CHUNK
        gather = pltpu.async_copy(table_hbm.at[idx_v.at[c]], pe_v, sem)
        pltpu.sync_copy(x_hbm.at[pl.ds(row0, _CHUNK)], in_v)
        gather.wait()

        @plsc.parallel_loop(0, _CHUNK)
        def _row(r):
            for j in range(n_vecs):
                sl = pl.ds(j * _LANES, _LANES)
                plsc.addupdate(in_v.at[r, sl], pe_v[r, sl])

        pltpu.sync_copy(in_v, out_hbm.at[pl.ds(row0, _CHUNK)])


@functools.partial(jax.jit, static_argnames=())
def kernel(inputs, inputs_positions, pos_embedding):
    b, s, d = inputs.shape
    n = b * s
    if inputs_positions is None:
        inputs_positions = jnp.broadcast_to(
            jnp.arange(s, dtype=jnp.int32)[None, :], (b, s))
    n_chunks = n // (_NW * _CHUNK)
    x = inputs.reshape(n, d)
    pos = inputs_positions.astype(jnp.int32).reshape(_NW, n_chunks, _CHUNK)
    table = pos_embedding.reshape(pos_embedding.shape[1], d)
    mesh = plsc.VectorSubcoreMesh(
        core_axis_name="c", subcore_axis_name="s",
        num_cores=_NC, num_subcores=_NS)
    out = pl.kernel(
        _sc_body,
        out_type=jax.ShapeDtypeStruct((n, d), jnp.float32),
        mesh=mesh,
        scratch_types=[
            pltpu.VMEM((n_chunks, _CHUNK), jnp.int32),
            pltpu.VMEM((_CHUNK, d), jnp.float32),
            pltpu.VMEM((_CHUNK, d), jnp.float32),
            pltpu.SemaphoreType.DMA,
        ],
    )(x, pos, table)
    return out.reshape(b, s, d)


# speedup vs baseline: 1.5377x; 1.5377x over previous
import functools

import jax
import jax.numpy as jnp
from jax import lax
from jax.experimental import pallas as pl
from jax.experimental.pallas import tpu as pltpu
from jax.experimental.pallas import tpu_sc as plsc

_LANES = 16
_NC, _NS = 2, 16
_NW = _NC * _NS
_CHUNK = 16


def _sc_body(x_hbm, pos_hbm, table_hbm, out_hbm,
             idx_v, in_v0, in_v1, pe_v0, pe_v1, gsem, lsem, ssem):
    wid = lax.axis_index("s") * _NC + lax.axis_index("c")
    n_chunks = pos_hbm.shape[1]
    d = x_hbm.shape[1]
    n_vecs = d // _LANES
    in_bufs = (in_v0, in_v1)
    pe_bufs = (pe_v0, pe_v1)

    def rows(c):
        return pl.ds((wid * n_chunks + c) * _CHUNK, _CHUNK)

    pltpu.sync_copy(pos_hbm.at[wid], idx_v)

    gathers = [None] * n_chunks
    loads = [None] * n_chunks
    stores = [None] * n_chunks
    gathers[0] = pltpu.async_copy(table_hbm.at[idx_v.at[0]], pe_bufs[0], gsem)
    loads[0] = pltpu.async_copy(x_hbm.at[rows(0)], in_bufs[0], lsem)
    for c in range(n_chunks):
        buf = c % 2
        nxt = (c + 1) % 2
        if c + 1 < n_chunks:
            if stores[c - 1] is not None:
                stores[c - 1].wait()
            gathers[c + 1] = pltpu.async_copy(
                table_hbm.at[idx_v.at[c + 1]], pe_bufs[nxt], gsem)
            loads[c + 1] = pltpu.async_copy(
                x_hbm.at[rows(c + 1)], in_bufs[nxt], lsem)
        gathers[c].wait()
        loads[c].wait()

        in_b = in_bufs[buf]
        pe_b = pe_bufs[buf]

        @plsc.parallel_loop(0, _CHUNK)
        def _row(r):
            @plsc.parallel_loop(0, d, _LANES, unroll=8)
            def _col(jj):
                sl = pl.ds(jj, _LANES)
                plsc.addupdate(in_b.at[r, sl], pe_b[r, sl])

        stores[c] = pltpu.async_copy(in_b, out_hbm.at[rows(c)], ssem)
    stores[n_chunks - 2].wait()
    stores[n_chunks - 1].wait()


@functools.partial(jax.jit, static_argnames=())
def kernel(inputs, inputs_positions, pos_embedding):
    b, s, d = inputs.shape
    n = b * s
    if inputs_positions is None:
        inputs_positions = jnp.broadcast_to(
            jnp.arange(s, dtype=jnp.int32)[None, :], (b, s))
    n_chunks = n // (_NW * _CHUNK)
    x = inputs.reshape(n, d)
    pos = inputs_positions.astype(jnp.int32).reshape(_NW, n_chunks, _CHUNK)
    table = pos_embedding.reshape(pos_embedding.shape[1], d)
    mesh = plsc.VectorSubcoreMesh(
        core_axis_name="c", subcore_axis_name="s",
        num_cores=_NC, num_subcores=_NS)
    out = pl.kernel(
        _sc_body,
        out_type=jax.ShapeDtypeStruct((n, d), jnp.float32),
        mesh=mesh,
        scratch_types=[
            pltpu.VMEM((n_chunks, _CHUNK), jnp.int32),
            pltpu.VMEM((_CHUNK, d), jnp.float32),
            pltpu.VMEM((_CHUNK, d), jnp.float32),
            pltpu.VMEM((_CHUNK, d), jnp.float32),
            pltpu.VMEM((_CHUNK, d), jnp.float32),
            pltpu.SemaphoreType.DMA,
            pltpu.SemaphoreType.DMA,
            pltpu.SemaphoreType.DMA,
        ],
    )(x, pos, table)
    return out.reshape(b, s, d)
